# baseline (speedup 1.0000x reference)
import functools
import jax
import jax.numpy as jnp
from jax import lax
from jax.experimental import pallas as pl
from jax.experimental.pallas import tpu as pltpu
from jax.experimental.pallas import tpu_sc as plsc

N = 10000
EDGES = 320000
D = 128
G = 256
VOCAB = 119

NPAD = 10240
NC, NS = 2, 16
NW = NC * NS
ECH = 96
CPW = 106
EPW = ECH * CPW
EPAD = NW * EPW
SLAB = NPAD // NS
GPW = G // NW
RCH = 32
RBLK = 512

_mesh = plsc.VectorSubcoreMesh(core_axis_name="c", subcore_axis_name="s",
                               num_cores=NC, num_subcores=NS)


@functools.partial(
    pl.kernel, mesh=_mesh,
    out_type=jax.ShapeDtypeStruct((NC, NPAD, D), jnp.float32),
    scratch_types=[
        pltpu.VMEM((EPW,), jnp.int32),
        pltpu.VMEM((EPW,), jnp.int32),
        pltpu.VMEM((ECH,), jnp.int32),
        pltpu.VMEM((ECH,), jnp.int32),
        pltpu.VMEM((ECH,), jnp.int32),
        pltpu.VMEM((ECH,), jnp.int32),
        pltpu.VMEM((ECH, D), jnp.float32),
        pltpu.VMEM((ECH, D), jnp.float32),
        pltpu.VMEM_SHARED((NPAD, D), jnp.float32),
        pltpu.SemaphoreType.DMA,
        pltpu.SemaphoreType.DMA,
        pltpu.SemaphoreType.DMA,
        pltpu.SemaphoreType.DMA,
    ],
)
def _seg_sum(h_hbm, src_hbm, dst_hbm, zeros_hbm, out_hbm,
             sidx, didx, sbuf0, sbuf1, dbuf0, dbuf1, rows0, rows1, agg,
             gsem0, gsem1, ssem0, ssem1):
    c = lax.axis_index("c")
    s = lax.axis_index("s")
    wid = s * NC + c
    pltpu.sync_copy(src_hbm.at[pl.ds(wid * EPW, EPW)], sidx)
    pltpu.sync_copy(dst_hbm.at[pl.ds(wid * EPW, EPW)], didx)
    pltpu.sync_copy(zeros_hbm.at[pl.ds(s * SLAB, SLAB)],
                    agg.at[pl.ds(s * SLAB, SLAB)])
    plsc.subcore_barrier()

    def cprow(buf, flat, k):
        for t in range(ECH // 16):
            buf[pl.ds(16 * t, 16)] = flat[pl.ds(k * ECH + 16 * t, 16)]

    def gstart(k, rbuf, sem, sbuf):
        cprow(sbuf, sidx, k)
        pltpu.async_copy(h_hbm.at[sbuf], rbuf, sem)

    def gwait(rbuf, sem, sbuf):
        pltpu.make_async_copy(h_hbm.at[sbuf], rbuf, sem).wait()

    def sstart(k, rbuf, dbuf, sem):
        cprow(dbuf, didx, k)
        pltpu.make_async_copy(rbuf, agg.at[dbuf], sem).start(add=True)

    def swait(rbuf, dbuf, sem):
        pltpu.make_async_copy(rbuf, agg.at[dbuf], sem).wait()

    gstart(0, rows0, gsem0, sbuf0)
    gstart(1, rows1, gsem1, sbuf1)
    gwait(rows0, gsem0, sbuf0)
    sstart(0, rows0, dbuf0, ssem0)

    def body(j, carry):
        k = 2 * j + 1
        gwait(rows1, gsem1, sbuf1)
        sstart(k, rows1, dbuf1, ssem1)
        swait(rows0, dbuf0, ssem0)
        gstart(k + 1, rows0, gsem0, sbuf0)
        gwait(rows0, gsem0, sbuf0)
        sstart(k + 1, rows0, dbuf0, ssem0)
        swait(rows1, dbuf1, ssem1)

        gstart(k + 2, rows1, gsem1, sbuf1)
        return carry

    lax.fori_loop(0, (CPW - 2) // 2, body, 0)
    gwait(rows1, gsem1, sbuf1)
    sstart(CPW - 1, rows1, dbuf1, ssem1)
    swait(rows0, dbuf0, ssem0)
    swait(rows1, dbuf1, ssem1)
    plsc.subcore_barrier()
    pltpu.sync_copy(agg.at[pl.ds(s * SLAB, SLAB)],
                    out_hbm.at[c, pl.ds(s * SLAB, SLAB)])


@functools.partial(
    pl.kernel, mesh=_mesh,
    out_type=jax.ShapeDtypeStruct((G * 3 * D,), jnp.float32),
    scratch_types=[
        pltpu.VMEM((NPAD,), jnp.int32),
        pltpu.VMEM((RCH * D,), jnp.float32),
        pltpu.VMEM((RCH * D,), jnp.float32),
        pltpu.VMEM((RCH * D,), jnp.float32),
        pltpu.VMEM((GPW * 3 * D,), jnp.float32),
    ],
)
def _pool(h1_hbm, h2_hbm, h3_hbm, batch_hbm, out_hbm, bv, r1, r2, r3, acc):
    c = lax.axis_index("c")
    s = lax.axis_index("s")
    wid = s * NC + c
    g0 = wid * GPW
    pltpu.sync_copy(batch_hbm, bv)

    def cbody(j, carry):
        lo_a, hi_a = carry
        chunk = bv[pl.ds(j * 16, 16)]
        lo_a = lo_a + jnp.where(chunk < g0, 1, 0)
        hi_a = hi_a + jnp.where(chunk < (g0 + GPW), 1, 0)
        return lo_a, hi_a

    z16 = jnp.zeros((16,), jnp.int32)
    lo_v, hi_v = lax.fori_loop(0, NPAD // 16, cbody, (z16, z16))
    lo = lo_v[0]
    hi = hi_v[0]
    for q in range(1, 16):
        lo = lo + lo_v[q]
        hi = hi + hi_v[q]

    ninf = jnp.full((16,), -jnp.inf, jnp.float32)
    for q in range(GPW * 3 * D // 16):
        acc[pl.ds(q * 16, 16)] = ninf

    lo = (lo // 16) * 16
    nch = (hi - lo + (RCH - 1)) // RCH

    def chunk_body(k, carry):
        start = jnp.minimum(lo + k * RCH, NPAD - RCH)
        pltpu.sync_copy(h1_hbm.at[pl.ds(start * D, RCH * D)], r1)
        pltpu.sync_copy(h2_hbm.at[pl.ds(start * D, RCH * D)], r2)
        pltpu.sync_copy(h3_hbm.at[pl.ds(start * D, RCH * D)], r3)

        for half in range(RCH // 16):
            gv = bv[pl.ds(start + half * 16, 16)] - g0
            for rr in range(16):
                r = half * 16 + rr
                g = gv[rr]

                @pl.when((g >= 0) & (g < GPW))
                def _(g=g, r=r):
                    ab = g * (3 * D)
                    rb = r * D
                    for cb in range(D // 16):
                        o = cb * 16
                        acc[pl.ds(ab + o, 16)] = jnp.maximum(
                            acc[pl.ds(ab + o, 16)], r1[pl.ds(rb + o, 16)])
                        acc[pl.ds(ab + D + o, 16)] = jnp.maximum(
                            acc[pl.ds(ab + D + o, 16)], r2[pl.ds(rb + o, 16)])
                        acc[pl.ds(ab + 2 * D + o, 16)] = jnp.maximum(
                            acc[pl.ds(ab + 2 * D + o, 16)], r3[pl.ds(rb + o, 16)])

        return carry

    lax.fori_loop(0, nch, chunk_body, 0)
    pltpu.sync_copy(acc, out_hbm.at[pl.ds(wid * GPW * 3 * D, GPW * 3 * D)])


def _emb_block(idx_ref, emb_ref, h_ref):
    ids = idx_ref[...]
    oh = (ids == lax.broadcasted_iota(jnp.int32, (RBLK, 128), 1))
    h_ref[...] = jnp.dot(oh.astype(jnp.float32), emb_ref[...],
                         preferred_element_type=jnp.float32,
                         precision=lax.Precision.HIGHEST)


_emb = pl.pallas_call(
    _emb_block,
    grid=(NPAD // RBLK,),
    in_specs=[
        pl.BlockSpec((RBLK, 1), lambda i: (i, 0)),
        pl.BlockSpec((128, D), lambda i: (0, 0)),
    ],
    out_specs=pl.BlockSpec((RBLK, D), lambda i: (i, 0)),
    out_shape=jax.ShapeDtypeStruct((NPAD, D), jnp.float32),
)


def _mlp_block(h_ref, agg_ref, w1_ref, b1_ref, w2_ref, b2_ref,
               z_ref, sums_ref):
    i = pl.program_id(0)
    xin = h_ref[...] + agg_ref[0] + agg_ref[1]
    z1 = jnp.maximum(
        jnp.dot(xin, w1_ref[...], preferred_element_type=jnp.float32,
                precision=lax.Precision.HIGHEST)
        + b1_ref[...], 0.0)
    z2 = jnp.maximum(
        jnp.dot(z1, w2_ref[...], preferred_element_type=jnp.float32,
                precision=lax.Precision.HIGHEST)
        + b2_ref[...], 0.0)
    z_ref[...] = z2
    rid = i * RBLK + lax.broadcasted_iota(jnp.int32, (RBLK, 1), 0)
    zm = jnp.where(rid < N, z2, 0.0)
    s1 = jnp.sum(zm, axis=0, keepdims=True)
    s2 = jnp.sum(zm * zm, axis=0, keepdims=True)
    contrib = jnp.concatenate([s1, s2, jnp.zeros((6, D), jnp.float32)], axis=0)

    @pl.when(i == 0)
    def _():
        sums_ref[...] = contrib

    @pl.when(i != 0)
    def _():
        sums_ref[...] = sums_ref[...] + contrib


_mlp = pl.pallas_call(
    _mlp_block,
    grid=(NPAD // RBLK,),
    in_specs=[
        pl.BlockSpec((RBLK, D), lambda i: (i, 0)),
        pl.BlockSpec((NC, RBLK, D), lambda i: (0, i, 0)),
        pl.BlockSpec((D, D), lambda i: (0, 0)),
        pl.BlockSpec((1, D), lambda i: (0, 0)),
        pl.BlockSpec((D, D), lambda i: (0, 0)),
        pl.BlockSpec((1, D), lambda i: (0, 0)),
    ],
    out_specs=[
        pl.BlockSpec((RBLK, D), lambda i: (i, 0)),
        pl.BlockSpec((8, D), lambda i: (0, 0)),
    ],
    out_shape=[
        jax.ShapeDtypeStruct((NPAD, D), jnp.float32),
        jax.ShapeDtypeStruct((8, D), jnp.float32),
    ],
)


def _norm_block(z_ref, sums_ref, g_ref, b_ref, h_ref):
    mean = sums_ref[0:1, :] * (1.0 / N)
    var = sums_ref[1:2, :] * (1.0 / N) - mean * mean
    rstd = lax.rsqrt(var + 1e-5)
    h_ref[...] = (z_ref[...] - mean) * (rstd * g_ref[...]) + b_ref[...]


_norm = pl.pallas_call(
    _norm_block,
    grid=(NPAD // RBLK,),
    in_specs=[
        pl.BlockSpec((RBLK, D), lambda i: (i, 0)),
        pl.BlockSpec((8, D), lambda i: (0, 0)),
        pl.BlockSpec((1, D), lambda i: (0, 0)),
        pl.BlockSpec((1, D), lambda i: (0, 0)),
    ],
    out_specs=pl.BlockSpec((RBLK, D), lambda i: (i, 0)),
    out_shape=jax.ShapeDtypeStruct((NPAD, D), jnp.float32),
)


def kernel(x, edge_index, batch, emb, W1, b1, W2, b2, gamma, beta):
    idx = jnp.concatenate(
        [x[:, 0].astype(jnp.int32), jnp.zeros((NPAD - N,), jnp.int32)])[:, None]
    src = jnp.concatenate(
        [edge_index[0].astype(jnp.int32),
         jnp.zeros((EPAD - EDGES,), jnp.int32)])
    dst = jnp.concatenate(
        [edge_index[1].astype(jnp.int32),
         jnp.full((EPAD - EDGES,), N, jnp.int32)])
    bpad = jnp.concatenate(
        [batch.astype(jnp.int32), jnp.full((NPAD - N,), G, jnp.int32)])
    emb_pad = jnp.zeros((128, D), jnp.float32).at[:VOCAB].set(emb)
    zeros_h = jnp.zeros((NPAD, D), jnp.float32)

    h = _emb(idx, emb_pad)
    hs = []
    for i in range(3):
        agg = _seg_sum(h, src, dst, zeros_h)
        z, sums = _mlp(h, agg, W1[i], b1[i][None, :], W2[i], b2[i][None, :])
        h = _norm(z, sums, gamma[i][None, :], beta[i][None, :])
        hs.append(h.reshape(-1))
    pooled = _pool(hs[0], hs[1], hs[2], bpad)
    return pooled.reshape(G, 3 * D)


# speedup vs baseline: 1.5271x; 1.5271x over previous
import functools
import jax
import jax.numpy as jnp
from jax import lax
from jax.experimental import pallas as pl
from jax.experimental.pallas import tpu as pltpu
from jax.experimental.pallas import tpu_sc as plsc

N = 10000
EDGES = 320000
D = 128
G = 256
VOCAB = 119

NPAD = 10240
NC, NS = 2, 16
NW = NC * NS
ECH = 96
CPW = 105
EPW = ECH * CPW
EPAD = NW * EPW
SLAB = NPAD // NS
GPW = G // NW
RCH = 32
RBLK = 512

_mesh = plsc.VectorSubcoreMesh(core_axis_name="c", subcore_axis_name="s",
                               num_cores=NC, num_subcores=NS)


@functools.partial(
    pl.kernel, mesh=_mesh,
    out_type=jax.ShapeDtypeStruct((NC, NPAD, D), jnp.float32),
    scratch_types=[
        pltpu.VMEM((EPW,), jnp.int32),
        pltpu.VMEM((EPW,), jnp.int32),
        pltpu.VMEM((ECH,), jnp.int32),
        pltpu.VMEM((ECH,), jnp.int32),
        pltpu.VMEM((ECH,), jnp.int32),
        pltpu.VMEM((ECH, D), jnp.float32),
        pltpu.VMEM((ECH, D), jnp.float32),
        pltpu.VMEM_SHARED((NPAD, D), jnp.float32),
        pltpu.SemaphoreType.DMA,
        pltpu.SemaphoreType.DMA,
    ],
)
def _seg_sum(h_hbm, src_hbm, dst_hbm, zeros_hbm, out_hbm,
             sidx, didx, sbuf0, sbuf1, dbuf, rows0, rows1, agg, sem0, sem1):
    c = lax.axis_index("c")
    s = lax.axis_index("s")
    wid = s * NC + c
    pltpu.sync_copy(src_hbm.at[pl.ds(wid * EPW, EPW)], sidx)
    pltpu.sync_copy(dst_hbm.at[pl.ds(wid * EPW, EPW)], didx)
    pltpu.sync_copy(zeros_hbm.at[pl.ds(s * SLAB, SLAB)],
                    agg.at[pl.ds(s * SLAB, SLAB)])
    plsc.subcore_barrier()

    def cprow(buf, flat, k):
        for t in range(ECH // 16):
            buf[pl.ds(16 * t, 16)] = flat[pl.ds(k * ECH + 16 * t, 16)]

    def gstart(k, rbuf, sem, sbuf):
        cprow(sbuf, sidx, k)
        pltpu.async_copy(h_hbm.at[sbuf], rbuf, sem)

    def gwait(rbuf, sem, sbuf):
        pltpu.make_async_copy(h_hbm.at[sbuf], rbuf, sem).wait()

    def scatter(k, rbuf):
        cprow(dbuf, didx, k)
        pltpu.sync_copy(rbuf, agg.at[dbuf], add=True)

    gstart(0, rows0, sem0, sbuf0)

    def body(j, carry):
        k = 2 * j
        gstart(k + 1, rows1, sem1, sbuf1)
        gwait(rows0, sem0, sbuf0)
        scatter(k, rows0)
        gstart(k + 2, rows0, sem0, sbuf0)
        gwait(rows1, sem1, sbuf1)
        scatter(k + 1, rows1)
        return carry

    lax.fori_loop(0, (CPW - 1) // 2, body, 0)
    gwait(rows0, sem0, sbuf0)
    scatter(CPW - 1, rows0)
    plsc.subcore_barrier()
    pltpu.sync_copy(agg.at[pl.ds(s * SLAB, SLAB)],
                    out_hbm.at[c, pl.ds(s * SLAB, SLAB)])


@functools.partial(
    pl.kernel, mesh=_mesh,
    out_type=jax.ShapeDtypeStruct((G * 3 * D,), jnp.float32),
    scratch_types=[
        pltpu.VMEM((NPAD,), jnp.int32),
        pltpu.VMEM((RCH * D,), jnp.float32),
        pltpu.VMEM((RCH * D,), jnp.float32),
        pltpu.VMEM((RCH * D,), jnp.float32),
        pltpu.VMEM((GPW * 3 * D,), jnp.float32),
    ],
)
def _pool(h1_hbm, h2_hbm, h3_hbm, batch_hbm, out_hbm, bv, r1, r2, r3, acc):
    c = lax.axis_index("c")
    s = lax.axis_index("s")
    wid = s * NC + c
    g0 = wid * GPW
    pltpu.sync_copy(batch_hbm, bv)

    def cbody(j, carry):
        lo_a, hi_a = carry
        chunk = bv[pl.ds(j * 16, 16)]
        lo_a = lo_a + jnp.where(chunk < g0, 1, 0)
        hi_a = hi_a + jnp.where(chunk < (g0 + GPW), 1, 0)
        return lo_a, hi_a

    z16 = jnp.zeros((16,), jnp.int32)
    lo_v, hi_v = lax.fori_loop(0, NPAD // 16, cbody, (z16, z16))
    lo = lo_v[0]
    hi = hi_v[0]
    for q in range(1, 16):
        lo = lo + lo_v[q]
        hi = hi + hi_v[q]

    ninf = jnp.full((16,), -jnp.inf, jnp.float32)
    for q in range(GPW * 3 * D // 16):
        acc[pl.ds(q * 16, 16)] = ninf

    lo = (lo // 16) * 16
    nch = (hi - lo + (RCH - 1)) // RCH

    def chunk_body(k, carry):
        start = jnp.minimum(lo + k * RCH, NPAD - RCH)
        pltpu.sync_copy(h1_hbm.at[pl.ds(start * D, RCH * D)], r1)
        pltpu.sync_copy(h2_hbm.at[pl.ds(start * D, RCH * D)], r2)
        pltpu.sync_copy(h3_hbm.at[pl.ds(start * D, RCH * D)], r3)

        for half in range(RCH // 16):
            gv = bv[pl.ds(start + half * 16, 16)] - g0
            for rr in range(16):
                r = half * 16 + rr
                g = gv[rr]

                @pl.when((g >= 0) & (g < GPW))
                def _(g=g, r=r):
                    ab = g * (3 * D)
                    rb = r * D
                    for cb in range(D // 16):
                        o = cb * 16
                        acc[pl.ds(ab + o, 16)] = jnp.maximum(
                            acc[pl.ds(ab + o, 16)], r1[pl.ds(rb + o, 16)])
                        acc[pl.ds(ab + D + o, 16)] = jnp.maximum(
                            acc[pl.ds(ab + D + o, 16)], r2[pl.ds(rb + o, 16)])
                        acc[pl.ds(ab + 2 * D + o, 16)] = jnp.maximum(
                            acc[pl.ds(ab + 2 * D + o, 16)], r3[pl.ds(rb + o, 16)])

        return carry

    lax.fori_loop(0, nch, chunk_body, 0)
    pltpu.sync_copy(acc, out_hbm.at[pl.ds(wid * GPW * 3 * D, GPW * 3 * D)])


def _emb_block(idx_ref, emb_ref, h_ref):
    ids = idx_ref[...]
    oh = (ids == lax.broadcasted_iota(jnp.int32, (RBLK, 128), 1))
    h_ref[...] = jnp.dot(oh.astype(jnp.float32), emb_ref[...],
                         preferred_element_type=jnp.float32,
                         precision=lax.Precision.HIGHEST)


_emb = pl.pallas_call(
    _emb_block,
    grid=(NPAD // RBLK,),
    in_specs=[
        pl.BlockSpec((RBLK, 1), lambda i: (i, 0)),
        pl.BlockSpec((128, D), lambda i: (0, 0)),
    ],
    out_specs=pl.BlockSpec((RBLK, D), lambda i: (i, 0)),
    out_shape=jax.ShapeDtypeStruct((NPAD, D), jnp.float32),
)


def _mlp_block(h_ref, agg_ref, w1_ref, b1_ref, w2_ref, b2_ref,
               z_ref, sums_ref):
    i = pl.program_id(0)
    xin = h_ref[...] + agg_ref[0] + agg_ref[1]
    z1 = jnp.maximum(
        jnp.dot(xin, w1_ref[...], preferred_element_type=jnp.float32,
                precision=lax.Precision.HIGHEST)
        + b1_ref[...], 0.0)
    z2 = jnp.maximum(
        jnp.dot(z1, w2_ref[...], preferred_element_type=jnp.float32,
                precision=lax.Precision.HIGHEST)
        + b2_ref[...], 0.0)
    z_ref[...] = z2
    rid = i * RBLK + lax.broadcasted_iota(jnp.int32, (RBLK, 1), 0)
    zm = jnp.where(rid < N, z2, 0.0)
    s1 = jnp.sum(zm, axis=0, keepdims=True)
    s2 = jnp.sum(zm * zm, axis=0, keepdims=True)
    contrib = jnp.concatenate([s1, s2, jnp.zeros((6, D), jnp.float32)], axis=0)

    @pl.when(i == 0)
    def _():
        sums_ref[...] = contrib

    @pl.when(i != 0)
    def _():
        sums_ref[...] = sums_ref[...] + contrib


_mlp = pl.pallas_call(
    _mlp_block,
    grid=(NPAD // RBLK,),
    in_specs=[
        pl.BlockSpec((RBLK, D), lambda i: (i, 0)),
        pl.BlockSpec((NC, RBLK, D), lambda i: (0, i, 0)),
        pl.BlockSpec((D, D), lambda i: (0, 0)),
        pl.BlockSpec((1, D), lambda i: (0, 0)),
        pl.BlockSpec((D, D), lambda i: (0, 0)),
        pl.BlockSpec((1, D), lambda i: (0, 0)),
    ],
    out_specs=[
        pl.BlockSpec((RBLK, D), lambda i: (i, 0)),
        pl.BlockSpec((8, D), lambda i: (0, 0)),
    ],
    out_shape=[
        jax.ShapeDtypeStruct((NPAD, D), jnp.float32),
        jax.ShapeDtypeStruct((8, D), jnp.float32),
    ],
)


def _norm_block(z_ref, sums_ref, g_ref, b_ref, h_ref):
    mean = sums_ref[0:1, :] * (1.0 / N)
    var = sums_ref[1:2, :] * (1.0 / N) - mean * mean
    rstd = lax.rsqrt(var + 1e-5)
    h_ref[...] = (z_ref[...] - mean) * (rstd * g_ref[...]) + b_ref[...]


_norm = pl.pallas_call(
    _norm_block,
    grid=(NPAD // RBLK,),
    in_specs=[
        pl.BlockSpec((RBLK, D), lambda i: (i, 0)),
        pl.BlockSpec((8, D), lambda i: (0, 0)),
        pl.BlockSpec((1, D), lambda i: (0, 0)),
        pl.BlockSpec((1, D), lambda i: (0, 0)),
    ],
    out_specs=pl.BlockSpec((RBLK, D), lambda i: (i, 0)),
    out_shape=jax.ShapeDtypeStruct((NPAD, D), jnp.float32),
)


def kernel(x, edge_index, batch, emb, W1, b1, W2, b2, gamma, beta):
    idx = jnp.concatenate(
        [x[:, 0].astype(jnp.int32), jnp.zeros((NPAD - N,), jnp.int32)])[:, None]
    src = jnp.concatenate(
        [edge_index[0].astype(jnp.int32),
         jnp.zeros((EPAD - EDGES,), jnp.int32)])
    dst = jnp.concatenate(
        [edge_index[1].astype(jnp.int32),
         jnp.full((EPAD - EDGES,), N, jnp.int32)])
    bpad = jnp.concatenate(
        [batch.astype(jnp.int32), jnp.full((NPAD - N,), G, jnp.int32)])
    emb_pad = jnp.zeros((128, D), jnp.float32).at[:VOCAB].set(emb)
    zeros_h = jnp.zeros((NPAD, D), jnp.float32)

    h = _emb(idx, emb_pad)
    hs = []
    for i in range(3):
        agg = _seg_sum(h, src, dst, zeros_h)
        z, sums = _mlp(h, agg, W1[i], b1[i][None, :], W2[i], b2[i][None, :])
        h = _norm(z, sums, gamma[i][None, :], beta[i][None, :])
        hs.append(h.reshape(-1))
    pooled = _pool(hs[0], hs[1], hs[2], bpad)
    return pooled.reshape(G, 3 * D)


# speedup vs baseline: 1.5501x; 1.0151x over previous
import functools
import jax
import jax.numpy as jnp
from jax import lax
from jax.experimental import pallas as pl
from jax.experimental.pallas import tpu as pltpu
from jax.experimental.pallas import tpu_sc as plsc

N = 10000
EDGES = 320000
D = 128
G = 256
VOCAB = 119

NPAD = 10240
NC, NS = 2, 16
NW = NC * NS
ECH = 96
CPW = 105
EPW = ECH * CPW
EPAD = NW * EPW
SLAB = NPAD // NS
GPW = G // NW
RCH = 32
RBLK = 512

_mesh = plsc.VectorSubcoreMesh(core_axis_name="c", subcore_axis_name="s",
                               num_cores=NC, num_subcores=NS)


@functools.partial(
    pl.kernel, mesh=_mesh,
    out_type=jax.ShapeDtypeStruct((NC, NPAD, D), jnp.float32),
    scratch_types=[
        pltpu.VMEM((EPW,), jnp.int32),
        pltpu.VMEM((EPW,), jnp.int32),
        pltpu.VMEM((ECH,), jnp.int32),
        pltpu.VMEM((ECH,), jnp.int32),
        pltpu.VMEM((ECH,), jnp.int32),
        pltpu.VMEM((ECH, D), jnp.float32),
        pltpu.VMEM((ECH, D), jnp.float32),
        pltpu.VMEM_SHARED((NPAD, D), jnp.float32),
        pltpu.SemaphoreType.DMA,
        pltpu.SemaphoreType.DMA,
    ],
)
def _seg_sum(h_hbm, src_hbm, dst_hbm, zeros_hbm, out_hbm,
             sidx, didx, sbuf0, sbuf1, dbuf, rows0, rows1, agg, sem0, sem1):
    c = lax.axis_index("c")
    s = lax.axis_index("s")
    wid = s * NC + c
    pltpu.sync_copy(src_hbm.at[pl.ds(wid * EPW, EPW)], sidx)
    pltpu.sync_copy(dst_hbm.at[pl.ds(wid * EPW, EPW)], didx)
    pltpu.sync_copy(zeros_hbm.at[pl.ds(s * SLAB, SLAB)],
                    agg.at[pl.ds(s * SLAB, SLAB)])
    plsc.subcore_barrier()

    def cprow(buf, flat, k):
        for t in range(ECH // 16):
            buf[pl.ds(16 * t, 16)] = flat[pl.ds(k * ECH + 16 * t, 16)]

    def gstart(k, rbuf, sem, sbuf):
        cprow(sbuf, sidx, k)
        pltpu.async_copy(h_hbm.at[sbuf], rbuf, sem)

    def gwait(rbuf, sem, sbuf):
        pltpu.make_async_copy(h_hbm.at[sbuf], rbuf, sem).wait()

    def scatter(k, rbuf):
        cprow(dbuf, didx, k)
        pltpu.sync_copy(rbuf, agg.at[dbuf], add=True)

    gstart(0, rows0, sem0, sbuf0)

    def body(j, carry):
        k = 2 * j
        gstart(k + 1, rows1, sem1, sbuf1)
        gwait(rows0, sem0, sbuf0)
        scatter(k, rows0)
        gstart(k + 2, rows0, sem0, sbuf0)
        gwait(rows1, sem1, sbuf1)
        scatter(k + 1, rows1)
        return carry

    lax.fori_loop(0, (CPW - 1) // 2, body, 0)
    gwait(rows0, sem0, sbuf0)
    scatter(CPW - 1, rows0)
    plsc.subcore_barrier()
    pltpu.sync_copy(agg.at[pl.ds(s * SLAB, SLAB)],
                    out_hbm.at[c, pl.ds(s * SLAB, SLAB)])


@functools.partial(
    pl.kernel, mesh=_mesh,
    out_type=jax.ShapeDtypeStruct((G * 3 * D,), jnp.float32),
    scratch_types=[
        pltpu.VMEM((NPAD,), jnp.int32),
        pltpu.VMEM((RCH * D,), jnp.float32),
        pltpu.VMEM((RCH * D,), jnp.float32),
        pltpu.VMEM((RCH * D,), jnp.float32),
        pltpu.VMEM((RCH * D,), jnp.float32),
        pltpu.VMEM((RCH * D,), jnp.float32),
        pltpu.VMEM((RCH * D,), jnp.float32),
        pltpu.VMEM((GPW * 3 * D,), jnp.float32),
        pltpu.SemaphoreType.DMA,
        pltpu.SemaphoreType.DMA,
        pltpu.SemaphoreType.DMA,
        pltpu.SemaphoreType.DMA,
        pltpu.SemaphoreType.DMA,
        pltpu.SemaphoreType.DMA,
    ],
)
def _pool(h1_hbm, h2_hbm, h3_hbm, batch_hbm, out_hbm, bv,
          r1a, r2a, r3a, r1b, r2b, r3b, acc,
          sa1, sa2, sa3, sb1, sb2, sb3):
    c = lax.axis_index("c")
    s = lax.axis_index("s")
    wid = s * NC + c
    g0 = wid * GPW
    pltpu.sync_copy(batch_hbm, bv)

    def cbody(j, carry):
        lo_a, hi_a = carry
        chunk = bv[pl.ds(j * 16, 16)]
        lo_a = lo_a + jnp.where(chunk < g0, 1, 0)
        hi_a = hi_a + jnp.where(chunk < (g0 + GPW), 1, 0)
        return lo_a, hi_a

    z16 = jnp.zeros((16,), jnp.int32)
    lo_v, hi_v = lax.fori_loop(0, NPAD // 16, cbody, (z16, z16))
    lo = lo_v[0]
    hi = hi_v[0]
    for q in range(1, 16):
        lo = lo + lo_v[q]
        hi = hi + hi_v[q]

    ninf = jnp.full((16,), -jnp.inf, jnp.float32)
    for q in range(GPW * 3 * D // 16):
        acc[pl.ds(q * 16, 16)] = ninf

    lo = (lo // 16) * 16
    nch = (hi - lo + (RCH - 1)) // RCH

    def cstart(k, b1, b2, b3, s1, s2, s3):
        start = jnp.minimum(lo + k * RCH, NPAD - RCH)
        pltpu.async_copy(h1_hbm.at[pl.ds(start * D, RCH * D)], b1, s1)
        pltpu.async_copy(h2_hbm.at[pl.ds(start * D, RCH * D)], b2, s2)
        pltpu.async_copy(h3_hbm.at[pl.ds(start * D, RCH * D)], b3, s3)

    def cwait(b1, b2, b3, s1, s2, s3):
        pltpu.make_async_copy(h1_hbm.at[pl.ds(0, RCH * D)], b1, s1).wait()
        pltpu.make_async_copy(h2_hbm.at[pl.ds(0, RCH * D)], b2, s2).wait()
        pltpu.make_async_copy(h3_hbm.at[pl.ds(0, RCH * D)], b3, s3).wait()

    def process(k, b1, b2, b3):
        start = jnp.minimum(lo + k * RCH, NPAD - RCH)
        for half in range(RCH // 16):
            gv = bv[pl.ds(start + half * 16, 16)] - g0
            for rr in range(16):
                r = half * 16 + rr
                g = gv[rr]

                @pl.when((g >= 0) & (g < GPW))
                def _(g=g, r=r):
                    ab = g * (3 * D)
                    rb = r * D
                    for cb in range(D // 16):
                        o = cb * 16
                        acc[pl.ds(ab + o, 16)] = jnp.maximum(
                            acc[pl.ds(ab + o, 16)], b1[pl.ds(rb + o, 16)])
                        acc[pl.ds(ab + D + o, 16)] = jnp.maximum(
                            acc[pl.ds(ab + D + o, 16)], b2[pl.ds(rb + o, 16)])
                        acc[pl.ds(ab + 2 * D + o, 16)] = jnp.maximum(
                            acc[pl.ds(ab + 2 * D + o, 16)], b3[pl.ds(rb + o, 16)])

    cstart(0, r1a, r2a, r3a, sa1, sa2, sa3)

    def chunk_body(k, carry):
        @pl.when(k % 2 == 0)
        def _():
            cwait(r1a, r2a, r3a, sa1, sa2, sa3)
            cstart(k + 1, r1b, r2b, r3b, sb1, sb2, sb3)
            process(k, r1a, r2a, r3a)

        @pl.when(k % 2 == 1)
        def _():
            cwait(r1b, r2b, r3b, sb1, sb2, sb3)
            cstart(k + 1, r1a, r2a, r3a, sa1, sa2, sa3)
            process(k, r1b, r2b, r3b)

        return carry

    lax.fori_loop(0, nch, chunk_body, 0)
    @pl.when(nch % 2 == 0)
    def _():
        cwait(r1a, r2a, r3a, sa1, sa2, sa3)

    @pl.when(nch % 2 == 1)
    def _():
        cwait(r1b, r2b, r3b, sb1, sb2, sb3)

    pltpu.sync_copy(acc, out_hbm.at[pl.ds(wid * GPW * 3 * D, GPW * 3 * D)])


def _emb_block(idx_ref, emb_ref, h_ref):
    ids = idx_ref[...]
    oh = (ids == lax.broadcasted_iota(jnp.int32, (RBLK, 128), 1))
    h_ref[...] = jnp.dot(oh.astype(jnp.float32), emb_ref[...],
                         preferred_element_type=jnp.float32,
                         precision=lax.Precision.HIGHEST)


_emb = pl.pallas_call(
    _emb_block,
    grid=(NPAD // RBLK,),
    in_specs=[
        pl.BlockSpec((RBLK, 1), lambda i: (i, 0)),
        pl.BlockSpec((128, D), lambda i: (0, 0)),
    ],
    out_specs=pl.BlockSpec((RBLK, D), lambda i: (i, 0)),
    out_shape=jax.ShapeDtypeStruct((NPAD, D), jnp.float32),
)


def _mlp_block(h_ref, agg_ref, w1_ref, b1_ref, w2_ref, b2_ref,
               z_ref, sums_ref):
    i = pl.program_id(0)
    xin = h_ref[...] + agg_ref[0] + agg_ref[1]
    z1 = jnp.maximum(
        jnp.dot(xin, w1_ref[...], preferred_element_type=jnp.float32,
                precision=lax.Precision.HIGHEST)
        + b1_ref[...], 0.0)
    z2 = jnp.maximum(
        jnp.dot(z1, w2_ref[...], preferred_element_type=jnp.float32,
                precision=lax.Precision.HIGHEST)
        + b2_ref[...], 0.0)
    z_ref[...] = z2
    rid = i * RBLK + lax.broadcasted_iota(jnp.int32, (RBLK, 1), 0)
    zm = jnp.where(rid < N, z2, 0.0)
    s1 = jnp.sum(zm, axis=0, keepdims=True)
    s2 = jnp.sum(zm * zm, axis=0, keepdims=True)
    contrib = jnp.concatenate([s1, s2, jnp.zeros((6, D), jnp.float32)], axis=0)

    @pl.when(i == 0)
    def _():
        sums_ref[...] = contrib

    @pl.when(i != 0)
    def _():
        sums_ref[...] = sums_ref[...] + contrib


_mlp = pl.pallas_call(
    _mlp_block,
    grid=(NPAD // RBLK,),
    in_specs=[
        pl.BlockSpec((RBLK, D), lambda i: (i, 0)),
        pl.BlockSpec((NC, RBLK, D), lambda i: (0, i, 0)),
        pl.BlockSpec((D, D), lambda i: (0, 0)),
        pl.BlockSpec((1, D), lambda i: (0, 0)),
        pl.BlockSpec((D, D), lambda i: (0, 0)),
        pl.BlockSpec((1, D), lambda i: (0, 0)),
    ],
    out_specs=[
        pl.BlockSpec((RBLK, D), lambda i: (i, 0)),
        pl.BlockSpec((8, D), lambda i: (0, 0)),
    ],
    out_shape=[
        jax.ShapeDtypeStruct((NPAD, D), jnp.float32),
        jax.ShapeDtypeStruct((8, D), jnp.float32),
    ],
)


def _norm_block(z_ref, sums_ref, g_ref, b_ref, h_ref):
    mean = sums_ref[0:1, :] * (1.0 / N)
    var = sums_ref[1:2, :] * (1.0 / N) - mean * mean
    rstd = lax.rsqrt(var + 1e-5)
    h_ref[...] = (z_ref[...] - mean) * (rstd * g_ref[...]) + b_ref[...]


_norm = pl.pallas_call(
    _norm_block,
    grid=(NPAD // RBLK,),
    in_specs=[
        pl.BlockSpec((RBLK, D), lambda i: (i, 0)),
        pl.BlockSpec((8, D), lambda i: (0, 0)),
        pl.BlockSpec((1, D), lambda i: (0, 0)),
        pl.BlockSpec((1, D), lambda i: (0, 0)),
    ],
    out_specs=pl.BlockSpec((RBLK, D), lambda i: (i, 0)),
    out_shape=jax.ShapeDtypeStruct((NPAD, D), jnp.float32),
)


def kernel(x, edge_index, batch, emb, W1, b1, W2, b2, gamma, beta):
    idx = jnp.concatenate(
        [x[:, 0].astype(jnp.int32), jnp.zeros((NPAD - N,), jnp.int32)])[:, None]
    src = jnp.concatenate(
        [edge_index[0].astype(jnp.int32),
         jnp.zeros((EPAD - EDGES,), jnp.int32)])
    dst = jnp.concatenate(
        [edge_index[1].astype(jnp.int32),
         jnp.full((EPAD - EDGES,), N, jnp.int32)])
    bpad = jnp.concatenate(
        [batch.astype(jnp.int32), jnp.full((NPAD - N,), G, jnp.int32)])
    emb_pad = jnp.zeros((128, D), jnp.float32).at[:VOCAB].set(emb)
    zeros_h = jnp.zeros((NPAD, D), jnp.float32)

    h = _emb(idx, emb_pad)
    hs = []
    for i in range(3):
        agg = _seg_sum(h, src, dst, zeros_h)
        z, sums = _mlp(h, agg, W1[i], b1[i][None, :], W2[i], b2[i][None, :])
        h = _norm(z, sums, gamma[i][None, :], beta[i][None, :])
        hs.append(h.reshape(-1))
    pooled = _pool(hs[0], hs[1], hs[2], bpad)
    return pooled.reshape(G, 3 * D)
